# baseline (speedup 1.0000x reference)
import functools

import jax
import jax.numpy as jnp
from jax import lax
from jax.experimental import pallas as pl
from jax.experimental.pallas import tpu as pltpu
from jax.experimental.pallas import tpu_sc as plsc

NC = 2
NS = 16
NW = NC * NS
CHUNK = 128
GK = 4
NBUF = 3
R_TC = 2048
TC_FRAC_NUM, TC_FRAC_DEN = 1, 2


def _sc_body_factory(n_rows, tc_rows, rows_per_w, n_blocks, n_emb):
    def body(table_hbm, idx_hbm, out_hbm, table_sh, idx_v, rows, sg, sw, sem0):
        wid = lax.axis_index("s") * NC + lax.axis_index("c")
        base = tc_rows + wid * rows_per_w
        blk = GK * CHUNK

        sid = lax.axis_index("s")
        pltpu.async_copy(
            table_hbm.at[pl.ds(1, n_emb - 1)],
            table_sh.at[pl.ds(sid * (n_emb - 1), n_emb - 1)], sem0).wait()
        pltpu.async_copy(idx_hbm.at[pl.ds(base, rows_per_w)], idx_v, sem0).wait()

        roff = sid * (n_emb - 1)

        def off_body(k, carry):
            sl = pl.ds(k * 16, 16)
            idx_v[sl] = idx_v[sl] + roff
            return carry

        lax.fori_loop(0, rows_per_w // 16, off_body, 0)

        def start_g(i, b):
            pltpu.async_copy(
                table_sh.at[idx_v.at[pl.ds(i * blk, blk)]], rows[b], sg[b])

        def wait_g(i, b):
            pltpu.make_async_copy(
                table_sh.at[idx_v.at[pl.ds(i * blk, blk)]], rows[b], sg[b]).wait()

        def start_w(i, b):
            pltpu.async_copy(
                rows[b], out_hbm.at[pl.ds(base + i * blk, blk)], sw[b])

        def wait_w(i, b):
            pltpu.make_async_copy(
                rows[b], out_hbm.at[pl.ds(base + i * blk, blk)], sw[b]).wait()

        start_g(0, 0)
        start_g(1, 1)
        for i in range(n_blocks):
            b = i % NBUF
            wait_g(i, b)
            if i >= 1:
                wait_w(i - 1, (i - 1) % NBUF)
            if i + 2 < n_blocks:
                start_g(i + 2, (i + 2) % NBUF)
            start_w(i, b)
        wait_w(n_blocks - 1, (n_blocks - 1) % NBUF)

    return body


@functools.partial(jax.jit, static_argnames=("n_rows", "d"))
def _lookup(idx_flat, table, *, n_rows, d):
    n_emb = table.shape[0]
    tc_rows = (n_rows * TC_FRAC_NUM // TC_FRAC_DEN) // R_TC * R_TC
    sc_rows = n_rows - tc_rows
    rows_per_w = sc_rows // NW
    n_chunks = rows_per_w // CHUNK
    n_blocks = n_chunks // GK
    assert n_blocks >= NBUF
    assert rows_per_w * NW == sc_rows
    assert n_blocks * GK * CHUNK == rows_per_w

    mesh = plsc.VectorSubcoreMesh(core_axis_name="c", subcore_axis_name="s")

    sc_call = functools.partial(
        pl.kernel,
        out_type=jax.ShapeDtypeStruct((n_rows, d), jnp.float32),
        mesh=mesh,
        scratch_types=[
            pltpu.VMEM_SHARED((NS * (n_emb - 1), d), jnp.float32),
            pltpu.VMEM((rows_per_w,), jnp.int32),
            tuple(pltpu.VMEM((GK * CHUNK, d), jnp.float32) for _ in range(NBUF)),
            tuple(pltpu.SemaphoreType.DMA for _ in range(NBUF)),
            tuple(pltpu.SemaphoreType.DMA for _ in range(NBUF)),
            pltpu.SemaphoreType.DMA,
        ],
        compiler_params=pltpu.CompilerParams(use_tc_tiling_on_sc=False),
    )(_sc_body_factory(n_rows, tc_rows, rows_per_w, n_blocks, n_emb))

    sc_out = sc_call(table, idx_flat)

    ir = R_TC // 128
    idx2d = idx_flat[:tc_rows].reshape(tc_rows // 128, 128)

    def tc_body(idx_ref, tbl_ref, _, out_ref):
        h2 = idx_ref[...]
        oh = (h2[:, :, None] + 1 == lax.broadcasted_iota(
            jnp.int32, (ir, 128, n_emb), 2)).astype(jnp.float32)
        out_ref[...] = jnp.dot(oh.reshape(R_TC, n_emb), tbl_ref[...],
                               precision=lax.Precision.HIGHEST,
                               preferred_element_type=jnp.float32)

    out = pl.pallas_call(
        tc_body,
        grid=(tc_rows // R_TC,),
        in_specs=[
            pl.BlockSpec((ir, 128), lambda i: (i, 0)),
            pl.BlockSpec((n_emb, d), lambda i: (0, 0)),
            pl.BlockSpec(memory_space=pl.ANY),
        ],
        out_specs=pl.BlockSpec((R_TC, d), lambda i: (i, 0)),
        out_shape=jax.ShapeDtypeStruct((n_rows, d), jnp.float32),
        input_output_aliases={2: 0},
    )(idx2d, table, sc_out)
    return out


def kernel(hop_distances, embedding_weight):
    b, t = hop_distances.shape
    _, d = embedding_weight.shape
    n_rows = b * t
    idx_flat = hop_distances.astype(jnp.int32).reshape(-1)
    out = _lookup(idx_flat, embedding_weight, n_rows=n_rows, d=d)
    return out.reshape(b, t, d)
